# baseline (speedup 1.0000x reference)
import functools

import jax
import jax.numpy as jnp
from jax import lax
from jax.experimental import pallas as pl
from jax.experimental.pallas import tpu as pltpu
from jax.experimental.pallas import tpu_sc as plsc

_VOCAB = 1000000
_EMBED_DIM = 32
_BATCH = 16384
_LANES = 16

_NUM_CORES = 2
_NUM_SUBCORES = 16
_NUM_WORKERS = _NUM_CORES * _NUM_SUBCORES
_B_PER_W = _BATCH // _NUM_WORKERS
_IDX_CHUNK = 128
_N_CHUNKS = _B_PER_W // _IDX_CHUNK
_PACK = 128 // _EMBED_DIM


@functools.partial(
    pl.kernel,
    mesh=plsc.VectorSubcoreMesh(core_axis_name="c", subcore_axis_name="s"),
    out_type=jax.ShapeDtypeStruct((_BATCH, _EMBED_DIM), jnp.float32),
    scratch_types=[
        pltpu.VMEM((_B_PER_W,), jnp.int32),
        pltpu.VMEM((_B_PER_W,), jnp.int32),
        pltpu.VMEM((2, _IDX_CHUNK, 128), jnp.float32),
        pltpu.VMEM((_B_PER_W, _EMBED_DIM), jnp.float32),
        pltpu.SemaphoreType.DMA,
    ],
)
def _gather_rows(table_hbm, idx_hbm, out_hbm, idx_v, widx_v, rows_v, out_v, sem):
    wid = lax.axis_index("s") * _NUM_CORES + lax.axis_index("c")
    base = wid * _B_PER_W
    pltpu.sync_copy(idx_hbm.at[pl.ds(base, _B_PER_W)], idx_v)

    def _widx_body(i, _):
        v = idx_v[pl.ds(i * _LANES, _LANES)]
        widx_v[pl.ds(i * _LANES, _LANES)] = lax.shift_right_logical(v, 2)
        return 0

    lax.fori_loop(0, _B_PER_W // _LANES, _widx_body, 0)

    def _fire(j):
        return pltpu.async_copy(
            table_hbm.at[widx_v.at[pl.ds(j * _IDX_CHUNK, _IDX_CHUNK)]],
            rows_v.at[j % 2],
            sem,
        )

    copies = [_fire(0)]
    for j in range(_N_CHUNKS):
        if j + 1 < _N_CHUNKS:
            copies.append(_fire(j + 1))
        copies[j].wait()
        buf = j % 2

        def _extract_group(g, _):
            v = idx_v[pl.ds(j * _IDX_CHUNK + g * _LANES, _LANES)]
            soff = (v & 3) * _EMBED_DIM
            for k in range(_LANES):
                row = g * _LANES + k
                s = soff[k]
                dst = j * _IDX_CHUNK + g * _LANES + k
                out_v[dst, pl.ds(0, _LANES)] = rows_v[buf, row, pl.ds(s, _LANES)]
                out_v[dst, pl.ds(_LANES, _LANES)] = rows_v[
                    buf, row, pl.ds(s + _LANES, _LANES)
                ]
            return 0

        lax.fori_loop(0, _IDX_CHUNK // _LANES, _extract_group, 0)

    pltpu.sync_copy(out_v, out_hbm.at[pl.ds(base, _B_PER_W)])


def kernel(table, indices):
    wide = table.reshape(_VOCAB // _PACK, 128)
    return _gather_rows(wide, indices.astype(jnp.int32))


# speedup vs baseline: 3.5693x; 3.5693x over previous
import functools

import jax
import jax.numpy as jnp
from jax import lax
from jax.experimental import pallas as pl
from jax.experimental.pallas import tpu as pltpu
from jax.experimental.pallas import tpu_sc as plsc

_VOCAB = 1000000
_EMBED_DIM = 32
_BATCH = 16384
_LANES = 16

_NUM_CORES = 2
_NUM_SUBCORES = 16
_NUM_WORKERS = _NUM_CORES * _NUM_SUBCORES
_B_PER_W = _BATCH // _NUM_WORKERS
_N_GROUPS = _B_PER_W // _LANES


@functools.partial(
    pl.kernel,
    mesh=plsc.VectorSubcoreMesh(core_axis_name="c", subcore_axis_name="s"),
    out_type=jax.ShapeDtypeStruct((_EMBED_DIM, _BATCH), jnp.float32),
    scratch_types=[
        pltpu.VMEM((_B_PER_W,), jnp.int32),
        pltpu.VMEM((_LANES, _EMBED_DIM, 128), jnp.float32),
        pltpu.VMEM((_EMBED_DIM, _B_PER_W), jnp.float32),
        pltpu.SemaphoreType.DMA,
    ],
    compiler_params=pltpu.CompilerParams(needs_layout_passes=False),
)
def _gather_cols(table_hbm, idx_hbm, out_hbm, idx_v, cols_v, obuf_v, sem):
    wid = lax.axis_index("s") * _NUM_CORES + lax.axis_index("c")
    base = wid * _B_PER_W
    pltpu.sync_copy(idx_hbm.at[pl.ds(base, _B_PER_W)], idx_v)

    iota16 = lax.iota(jnp.int32, _LANES)

    def _group(g, _):
        v = idx_v[pl.ds(g * _LANES, _LANES)]
        bvec = lax.shift_right_logical(v, 7)
        lvec = v & 127
        copies = []
        for k in range(_LANES):
            boff = pl.multiple_of(bvec[k] * 128, 128)
            copies.append(
                pltpu.async_copy(
                    table_hbm.at[:, pl.ds(boff, 128)],
                    cols_v.at[k],
                    sem,
                )
            )
        for c in copies:
            c.wait()
        for k in range(_LANES):
            lcol = jnp.full((_LANES,), lvec[k], jnp.int32)
            dstc = jnp.full((_LANES,), g * _LANES + k, jnp.int32)
            lo = plsc.load_gather(cols_v.at[k], [iota16, lcol])
            hi = plsc.load_gather(cols_v.at[k], [iota16 + _LANES, lcol])
            plsc.store_scatter(obuf_v, [iota16, dstc], lo)
            plsc.store_scatter(obuf_v, [iota16 + _LANES, dstc], hi)
        return 0

    lax.fori_loop(0, _N_GROUPS, _group, 0)

    pltpu.sync_copy(obuf_v, out_hbm.at[:, pl.ds(base, _B_PER_W)])


def kernel(table, indices):
    out_t = _gather_cols(table.T, indices.astype(jnp.int32))
    return out_t.T
